# baseline (speedup 1.0000x reference)
import functools

import jax
import jax.numpy as jnp
from jax import lax
from jax.experimental import pallas as pl
from jax.experimental.pallas import tpu as pltpu
from jax.experimental.pallas import tpu_sc as plsc

_VOCAB = 30522
_MAX_POS = 512
_N_TYPES = 2
_D = 768
_B = 128
_S = 512
_EPS = 1e-12

_L = 16
_NC = 2
_NS = 16
_NW = _NC * _NS
_S_PER_W = _S // _NW
_CH = _D // _L


def _bcast_sum(v):
    idx = lax.iota(jnp.int32, _L)
    for sh in (8, 4, 2, 1):
        perm = jnp.bitwise_xor(idx, sh)
        v = v + v.at[perm].get(mode="promise_in_bounds")
    return v


def _rsqrt_newton(v):
    iv = lax.bitcast_convert_type(v, jnp.int32)
    y = lax.bitcast_convert_type(jnp.int32(0x5F3759DF) - (iv >> 1), jnp.float32)
    for _ in range(3):
        y = y * (1.5 - 0.5 * v * y * y)
    return y


def _make_sc_kernel():
    mesh = plsc.VectorSubcoreMesh(core_axis_name="c", subcore_axis_name="s")

    @functools.partial(
        pl.kernel,
        mesh=mesh,
        out_type=jax.ShapeDtypeStruct((_B, _S, _D), jnp.float32),
        scratch_types=[
            pltpu.VMEM((_S_PER_W, _D), jnp.float32),
            pltpu.VMEM((_D,), jnp.float32),
            pltpu.VMEM((_D,), jnp.float32),
            pltpu.VMEM((_D,), jnp.float32),
            pltpu.VMEM((_N_TYPES, _D), jnp.float32),
            pltpu.VMEM((_L,), jnp.int32),
            pltpu.VMEM((_L,), jnp.int32),
            pltpu.VMEM((_L, _D), jnp.float32),
            pltpu.SemaphoreType.DMA,
        ],
    )
    def emb_kernel(ids_hbm, tt_hbm, word_hbm, pos_hbm, type_hbm, g_hbm, b_hbm,
                   out_hbm, base_v, dt_v, g_v, b_v, type_v, ids_v, tt_v,
                   rows_v, sem):
        wid = lax.axis_index("s") * _NC + lax.axis_index("c")
        s0 = pl.multiple_of(wid * _S_PER_W, _S_PER_W)

        pltpu.sync_copy(pos_hbm.at[pl.ds(s0, _S_PER_W)], base_v)
        pltpu.sync_copy(type_hbm, type_v)
        pltpu.sync_copy(g_hbm, g_v)
        pltpu.sync_copy(b_hbm, b_v)

        for j in range(_CH):
            off = j * _L
            dt_v[pl.ds(off, _L)] = type_v[1, pl.ds(off, _L)] - type_v[0, pl.ds(off, _L)]

        def _add_t0(i, _):
            for j in range(_CH):
                off = j * _L
                base_v[i, pl.ds(off, _L)] = (
                    base_v[i, pl.ds(off, _L)] + type_v[0, pl.ds(off, _L)])
            return 0
        lax.fori_loop(0, _S_PER_W, _add_t0, 0)

        inv_d = jnp.float32(1.0 / _D)

        def _per_batch(bb, _):
            pltpu.sync_copy(ids_hbm.at[bb, pl.ds(s0, _S_PER_W)], ids_v)
            pltpu.sync_copy(tt_hbm.at[bb, pl.ds(s0, _S_PER_W)], tt_v)
            pltpu.async_copy(word_hbm.at[ids_v], rows_v, sem).wait()

            tt_vec = tt_v[...].astype(jnp.float32)

            def _per_token(i, _):
                ttf = tt_vec.at[jnp.full((_L,), i, jnp.int32)].get(
                    mode="promise_in_bounds")

                def _stats(j, carry):
                    ssum, ssq = carry
                    off = pl.multiple_of(j * _L, _L)
                    x = (rows_v[i, pl.ds(off, _L)]
                         + base_v[i, pl.ds(off, _L)]
                         + ttf * dt_v[pl.ds(off, _L)])
                    rows_v[i, pl.ds(off, _L)] = x
                    return ssum + x, ssq + x * x
                ssum, ssq = lax.fori_loop(
                    0, _CH, _stats,
                    (jnp.zeros((_L,), jnp.float32), jnp.zeros((_L,), jnp.float32)))

                mvec = _bcast_sum(ssum) * inv_d
                vvec = _bcast_sum(ssq) * inv_d - mvec * mvec
                rvec = _rsqrt_newton(vvec + jnp.float32(_EPS))

                def _norm(j, _):
                    off = pl.multiple_of(j * _L, _L)
                    x = rows_v[i, pl.ds(off, _L)]
                    y = (x - mvec) * rvec
                    rows_v[i, pl.ds(off, _L)] = y * g_v[pl.ds(off, _L)] + b_v[pl.ds(off, _L)]
                    return 0
                lax.fori_loop(0, _CH, _norm, 0)
                return 0
            lax.fori_loop(0, _L, _per_token, 0)

            pltpu.sync_copy(rows_v, out_hbm.at[bb, pl.ds(s0, _S_PER_W)])
            return 0
        lax.fori_loop(0, _B, _per_batch, 0)

    return emb_kernel


_EMB_KERNEL = _make_sc_kernel()


def kernel(input_ids, token_type_ids, word_emb, pos_emb, type_emb, ln_gamma,
           ln_beta):
    ids = input_ids.astype(jnp.int32)
    tt = token_type_ids.astype(jnp.int32)
    return _EMB_KERNEL(ids, tt, word_emb, pos_emb, type_emb, ln_gamma, ln_beta)


# speedup vs baseline: 2.1235x; 2.1235x over previous
import functools

import jax
import jax.numpy as jnp
from jax import lax
from jax.experimental import pallas as pl
from jax.experimental.pallas import tpu as pltpu
from jax.experimental.pallas import tpu_sc as plsc

_VOCAB = 30522
_MAX_POS = 512
_N_TYPES = 2
_D = 768
_B = 128
_S = 512
_EPS = 1e-12

_L = 16
_NC = 2
_NS = 16
_NW = _NC * _NS
_S_PER_W = _S // _NW
_CH = _D // _L


def _bcast_sum(v):
    idx = lax.iota(jnp.int32, _L)
    for sh in (8, 4, 2, 1):
        perm = jnp.bitwise_xor(idx, sh)
        v = v + v.at[perm].get(mode="promise_in_bounds")
    return v


def _rsqrt_newton(v):
    iv = lax.bitcast_convert_type(v, jnp.int32)
    y = lax.bitcast_convert_type(jnp.int32(0x5F3759DF) - (iv >> 1), jnp.float32)
    for _ in range(3):
        y = y * (1.5 - 0.5 * v * y * y)
    return y


def _make_sc_kernel():
    mesh = plsc.VectorSubcoreMesh(core_axis_name="c", subcore_axis_name="s")

    @functools.partial(
        pl.kernel,
        mesh=mesh,
        out_type=jax.ShapeDtypeStruct((_B, _S, _D), jnp.float32),
        scratch_types=[
            pltpu.VMEM((_S_PER_W, _D), jnp.float32),
            pltpu.VMEM((_D,), jnp.float32),
            pltpu.VMEM((_D,), jnp.float32),
            pltpu.VMEM((_D,), jnp.float32),
            pltpu.VMEM((_N_TYPES, _D), jnp.float32),
            pltpu.VMEM((_B, _S_PER_W), jnp.int32),
            pltpu.VMEM((_B, _S_PER_W), jnp.int32),
            pltpu.VMEM((_L, _D), jnp.float32),
            pltpu.VMEM((_L, _D), jnp.float32),
            pltpu.SemaphoreType.DMA,
            pltpu.SemaphoreType.DMA,
        ],
    )
    def emb_kernel(ids_hbm, tt_hbm, word_hbm, pos_hbm, type_hbm, g_hbm, b_hbm,
                   out_hbm, base_v, dt_v, g_v, b_v, type_v, ids_all, tt_all,
                   rows0, rows1, sem0, sem1):
        wid = lax.axis_index("s") * _NC + lax.axis_index("c")
        s0 = pl.multiple_of(wid * _S_PER_W, _S_PER_W)

        pltpu.sync_copy(pos_hbm.at[pl.ds(s0, _S_PER_W)], base_v)
        pltpu.sync_copy(type_hbm, type_v)
        pltpu.sync_copy(g_hbm, g_v)
        pltpu.sync_copy(b_hbm, b_v)
        pltpu.sync_copy(ids_hbm.at[wid], ids_all)
        pltpu.sync_copy(tt_hbm.at[wid], tt_all)

        for j in range(_CH):
            off = j * _L
            dt_v[pl.ds(off, _L)] = type_v[1, pl.ds(off, _L)] - type_v[0, pl.ds(off, _L)]

        def _add_t0(i, _):
            for j in range(_CH):
                off = j * _L
                base_v[i, pl.ds(off, _L)] = (
                    base_v[i, pl.ds(off, _L)] + type_v[0, pl.ds(off, _L)])
            return 0
        lax.fori_loop(0, _S_PER_W, _add_t0, 0)

        inv_d = jnp.float32(1.0 / _D)

        def _compute(rows_ref, bb):
            tt_vec = tt_all[bb, :].astype(jnp.float32)

            def _per_token(i, _):
                ttf = tt_vec.at[jnp.full((_L,), i, jnp.int32)].get(
                    mode="promise_in_bounds")
                ssum = jnp.zeros((_L,), jnp.float32)
                ssq = jnp.zeros((_L,), jnp.float32)
                xs = []
                for j in range(_CH):
                    off = j * _L
                    x = (rows_ref[i, pl.ds(off, _L)]
                         + base_v[i, pl.ds(off, _L)]
                         + ttf * dt_v[pl.ds(off, _L)])
                    xs.append(x)
                    ssum = ssum + x
                    ssq = ssq + x * x
                mvec = _bcast_sum(ssum) * inv_d
                vvec = _bcast_sum(ssq) * inv_d - mvec * mvec
                rvec = _rsqrt_newton(vvec + jnp.float32(_EPS))
                nmr = -mvec * rvec
                for j in range(_CH):
                    off = j * _L
                    y = xs[j] * rvec + nmr
                    rows_ref[i, pl.ds(off, _L)] = (
                        y * g_v[pl.ds(off, _L)] + b_v[pl.ds(off, _L)])
                return 0
            lax.fori_loop(0, _L, _per_token, 0)

        pltpu.async_copy(word_hbm.at[ids_all.at[0]], rows0, sem0)

        def _pair(k, _):
            b0 = 2 * k
            b1 = b0 + 1
            pltpu.async_copy(word_hbm.at[ids_all.at[b1]], rows1, sem1)
            pltpu.make_async_copy(word_hbm.at[ids_all.at[b0]], rows0, sem0).wait()
            _compute(rows0, b0)
            pltpu.sync_copy(rows0, out_hbm.at[b0, pl.ds(s0, _S_PER_W)])

            @pl.when(k < _B // 2 - 1)
            def _():
                pltpu.async_copy(word_hbm.at[ids_all.at[b0 + 2]], rows0, sem0)

            pltpu.make_async_copy(word_hbm.at[ids_all.at[b1]], rows1, sem1).wait()
            _compute(rows1, b1)
            pltpu.sync_copy(rows1, out_hbm.at[b1, pl.ds(s0, _S_PER_W)])
            return 0
        lax.fori_loop(0, _B // 2, _pair, 0)

    return emb_kernel


_EMB_KERNEL = _make_sc_kernel()


def kernel(input_ids, token_type_ids, word_emb, pos_emb, type_emb, ln_gamma,
           ln_beta):
    ids = (input_ids.astype(jnp.int32)
           .reshape(_B, _NW, _S_PER_W).transpose(1, 0, 2))
    tt = (token_type_ids.astype(jnp.int32)
          .reshape(_B, _NW, _S_PER_W).transpose(1, 0, 2))
    return _EMB_KERNEL(ids, tt, word_emb, pos_emb, type_emb, ln_gamma, ln_beta)
